# baseline (speedup 1.0000x reference)
import jax
import jax.numpy as jnp
from jax.experimental import pallas as pl
from jax.experimental.pallas import tpu as pltpu

NUM_POINTS = 2_000_000
N_NEW = 1_000_000
N_FEAT = 1000
D_FEAT = 512

TILE = 128
P_LO = (N_NEW // TILE) * TILE
P_HI = P_LO + TILE


def _copy_body(np_ref, bnd_ref, lbnd_ref, pose_ref, nl_ref, feat_ref,
               pbuf_ref, lbuf_ref, out_p, out_pose, out_l, out_f, sem):
    copies = [
        pltpu.make_async_copy(np_ref.at[pl.ds(0, P_LO)],
                              out_p.at[pl.ds(0, P_LO)], sem),
        pltpu.make_async_copy(bnd_ref, out_p.at[pl.ds(P_LO, TILE)], sem),
        pltpu.make_async_copy(pbuf_ref.at[pl.ds(P_HI, NUM_POINTS - P_HI)],
                              out_p.at[pl.ds(P_HI, NUM_POINTS - P_HI)], sem),
        pltpu.make_async_copy(nl_ref.at[pl.ds(0, P_LO)],
                              out_l.at[pl.ds(0, P_LO)], sem),
        pltpu.make_async_copy(lbnd_ref, out_l.at[pl.ds(P_LO, TILE)], sem),
        pltpu.make_async_copy(lbuf_ref.at[pl.ds(P_HI, NUM_POINTS - P_HI)],
                              out_l.at[pl.ds(P_HI, NUM_POINTS - P_HI)], sem),
        pltpu.make_async_copy(feat_ref, out_f, sem),
        pltpu.make_async_copy(pose_ref, out_pose, sem),
    ]
    for c in copies:
        c.start()
    for c in copies:
        c.wait()


def kernel(new_points, pose, new_point_label, new_label_feature,
           points_buf, points_label_buf, label_feature_buf, pose_buf):
    del label_feature_buf, pose_buf
    boundary = jnp.concatenate(
        [new_points[P_LO:], points_buf[N_NEW:P_HI]], axis=0)
    lboundary = jnp.concatenate(
        [new_point_label[P_LO:], points_label_buf[N_NEW:P_HI]], axis=0)
    return pl.pallas_call(
        _copy_body,
        in_specs=[pl.BlockSpec(memory_space=pl.ANY)] * 8,
        out_specs=[pl.BlockSpec(memory_space=pl.ANY)] * 4,
        out_shape=(
            jax.ShapeDtypeStruct((NUM_POINTS, 3), jnp.float32),
            jax.ShapeDtypeStruct((4, 4), jnp.float32),
            jax.ShapeDtypeStruct((NUM_POINTS,), jnp.int32),
            jax.ShapeDtypeStruct((N_FEAT, D_FEAT), jnp.float32),
        ),
        scratch_shapes=[pltpu.SemaphoreType.DMA],
    )(new_points, boundary, lboundary, pose, new_point_label,
      new_label_feature, points_buf, points_label_buf)


# speedup vs baseline: 16.9636x; 16.9636x over previous
import jax
import jax.numpy as jnp
from jax.experimental import pallas as pl
from jax.experimental.pallas import tpu as pltpu

NUM_POINTS = 2_000_000
N_NEW = 1_000_000
N_FEAT = 1000
D_FEAT = 512

B = 8192
GRID = (NUM_POINTS + B - 1) // B
NBB = N_NEW // B


def _copy_body(np_blk, pbuf_blk, nl_blk, lbuf_blk, feat_ref, pose_ref,
               out_p, out_l, out_f, out_pose, sem):
    i = pl.program_id(0)
    row0 = i * B
    pmask = jax.lax.broadcasted_iota(jnp.int32, (B, 1), 0) + row0 < N_NEW
    out_p[...] = jnp.where(pmask, np_blk[...], pbuf_blk[...])
    lmask = jax.lax.broadcasted_iota(jnp.int32, (B,), 0) + row0 < N_NEW
    out_l[...] = jnp.where(lmask, nl_blk[...], lbuf_blk[...])

    @pl.when(i == 0)
    def _small():
        cf = pltpu.make_async_copy(feat_ref, out_f, sem)
        cf.start()
        cf.wait()
        cp = pltpu.make_async_copy(pose_ref, out_pose, sem)
        cp.start()
        cp.wait()


def kernel(new_points, pose, new_point_label, new_label_feature,
           points_buf, points_label_buf, label_feature_buf, pose_buf):
    del label_feature_buf, pose_buf
    out_p, out_l, out_f, out_pose = pl.pallas_call(
        _copy_body,
        grid=GRID,
        in_specs=[
            pl.BlockSpec((B, 3), lambda i: (jnp.minimum(i, NBB), 0)),
            pl.BlockSpec((B, 3), lambda i: (jnp.maximum(i, NBB), 0)),
            pl.BlockSpec((B,), lambda i: (jnp.minimum(i, NBB),)),
            pl.BlockSpec((B,), lambda i: (jnp.maximum(i, NBB),)),
            pl.BlockSpec(memory_space=pl.ANY),
            pl.BlockSpec(memory_space=pl.ANY),
        ],
        out_specs=(
            pl.BlockSpec((B, 3), lambda i: (i, 0)),
            pl.BlockSpec((B,), lambda i: (i,)),
            pl.BlockSpec(memory_space=pl.ANY),
            pl.BlockSpec(memory_space=pl.ANY),
        ),
        out_shape=(
            jax.ShapeDtypeStruct((NUM_POINTS, 3), jnp.float32),
            jax.ShapeDtypeStruct((NUM_POINTS,), jnp.int32),
            jax.ShapeDtypeStruct((N_FEAT, D_FEAT), jnp.float32),
            jax.ShapeDtypeStruct((4, 4), jnp.float32),
        ),
        scratch_shapes=[pltpu.SemaphoreType.DMA],
    )(new_points, points_buf, new_point_label, points_label_buf,
      new_label_feature, pose)
    return out_p, out_pose, out_l, out_f


# speedup vs baseline: 16.9693x; 1.0003x over previous
import jax
import jax.numpy as jnp
from jax.experimental import pallas as pl
from jax.experimental.pallas import tpu as pltpu

NUM_POINTS = 2_000_000
N_NEW = 1_000_000
N_FEAT = 1000
D_FEAT = 512

B = 8192
GRID = (NUM_POINTS + B - 1) // B
NBB = N_NEW // B


def _copy_body(np_blk, pbuf_blk, nl_blk, lbuf_blk, feat_ref, pose_ref,
               out_p, out_l, out_f, out_pose, sem):
    i = pl.program_id(0)
    row0 = i * B

    @pl.when(i < NBB)
    def _first_half():
        out_p[...] = np_blk[...]
        out_l[...] = nl_blk[...]

    @pl.when(i == NBB)
    def _boundary():
        pmask = jax.lax.broadcasted_iota(jnp.int32, (B, 1), 0) + row0 < N_NEW
        out_p[...] = jnp.where(pmask, np_blk[...], pbuf_blk[...])
        lmask = jax.lax.broadcasted_iota(jnp.int32, (B,), 0) + row0 < N_NEW
        out_l[...] = jnp.where(lmask, nl_blk[...], lbuf_blk[...])

    @pl.when(i > NBB)
    def _second_half():
        out_p[...] = pbuf_blk[...]
        out_l[...] = lbuf_blk[...]

    @pl.when(i == 0)
    def _small():
        cf = pltpu.make_async_copy(feat_ref, out_f, sem)
        cf.start()
        cf.wait()
        cp = pltpu.make_async_copy(pose_ref, out_pose, sem)
        cp.start()
        cp.wait()


def kernel(new_points, pose, new_point_label, new_label_feature,
           points_buf, points_label_buf, label_feature_buf, pose_buf):
    del label_feature_buf, pose_buf
    out_p, out_l, out_f, out_pose = pl.pallas_call(
        _copy_body,
        grid=GRID,
        in_specs=[
            pl.BlockSpec((B, 3), lambda i: (jnp.minimum(i, NBB), 0)),
            pl.BlockSpec((B, 3), lambda i: (jnp.maximum(i, NBB), 0)),
            pl.BlockSpec((B,), lambda i: (jnp.minimum(i, NBB),)),
            pl.BlockSpec((B,), lambda i: (jnp.maximum(i, NBB),)),
            pl.BlockSpec(memory_space=pl.ANY),
            pl.BlockSpec(memory_space=pl.ANY),
        ],
        out_specs=(
            pl.BlockSpec((B, 3), lambda i: (i, 0)),
            pl.BlockSpec((B,), lambda i: (i,)),
            pl.BlockSpec(memory_space=pl.ANY),
            pl.BlockSpec(memory_space=pl.ANY),
        ),
        out_shape=(
            jax.ShapeDtypeStruct((NUM_POINTS, 3), jnp.float32),
            jax.ShapeDtypeStruct((NUM_POINTS,), jnp.int32),
            jax.ShapeDtypeStruct((N_FEAT, D_FEAT), jnp.float32),
            jax.ShapeDtypeStruct((4, 4), jnp.float32),
        ),
        scratch_shapes=[pltpu.SemaphoreType.DMA],
    )(new_points, points_buf, new_point_label, points_label_buf,
      new_label_feature, pose)
    return out_p, out_pose, out_l, out_f


# speedup vs baseline: 17.0293x; 1.0035x over previous
import jax
import jax.numpy as jnp
from jax.experimental import pallas as pl
from jax.experimental.pallas import tpu as pltpu

NUM_POINTS = 2_000_000
N_NEW = 1_000_000
N_FEAT = 1000
D_FEAT = 512

B = 16384
GRID = (NUM_POINTS + B - 1) // B
NBB = N_NEW // B


def _copy_body(np_blk, pbuf_blk, nl_blk, lbuf_blk, feat_ref, pose_ref,
               out_p, out_l, out_f, out_pose, sem):
    i = pl.program_id(0)
    row0 = i * B

    @pl.when(i < NBB)
    def _first_half():
        out_p[...] = np_blk[...]
        out_l[...] = nl_blk[...]

    @pl.when(i == NBB)
    def _boundary():
        pmask = jax.lax.broadcasted_iota(jnp.int32, (B, 1), 0) + row0 < N_NEW
        out_p[...] = jnp.where(pmask, np_blk[...], pbuf_blk[...])
        lmask = jax.lax.broadcasted_iota(jnp.int32, (B,), 0) + row0 < N_NEW
        out_l[...] = jnp.where(lmask, nl_blk[...], lbuf_blk[...])

    @pl.when(i > NBB)
    def _second_half():
        out_p[...] = pbuf_blk[...]
        out_l[...] = lbuf_blk[...]

    @pl.when(i == 0)
    def _small():
        cf = pltpu.make_async_copy(feat_ref, out_f, sem)
        cf.start()
        cf.wait()
        cp = pltpu.make_async_copy(pose_ref, out_pose, sem)
        cp.start()
        cp.wait()


def kernel(new_points, pose, new_point_label, new_label_feature,
           points_buf, points_label_buf, label_feature_buf, pose_buf):
    del label_feature_buf, pose_buf
    out_p, out_l, out_f, out_pose = pl.pallas_call(
        _copy_body,
        grid=GRID,
        in_specs=[
            pl.BlockSpec((B, 3), lambda i: (jnp.minimum(i, NBB), 0)),
            pl.BlockSpec((B, 3), lambda i: (jnp.maximum(i, NBB), 0)),
            pl.BlockSpec((B,), lambda i: (jnp.minimum(i, NBB),)),
            pl.BlockSpec((B,), lambda i: (jnp.maximum(i, NBB),)),
            pl.BlockSpec(memory_space=pl.ANY),
            pl.BlockSpec(memory_space=pl.ANY),
        ],
        out_specs=(
            pl.BlockSpec((B, 3), lambda i: (i, 0)),
            pl.BlockSpec((B,), lambda i: (i,)),
            pl.BlockSpec(memory_space=pl.ANY),
            pl.BlockSpec(memory_space=pl.ANY),
        ),
        out_shape=(
            jax.ShapeDtypeStruct((NUM_POINTS, 3), jnp.float32),
            jax.ShapeDtypeStruct((NUM_POINTS,), jnp.int32),
            jax.ShapeDtypeStruct((N_FEAT, D_FEAT), jnp.float32),
            jax.ShapeDtypeStruct((4, 4), jnp.float32),
        ),
        scratch_shapes=[pltpu.SemaphoreType.DMA],
    )(new_points, points_buf, new_point_label, points_label_buf,
      new_label_feature, pose)
    return out_p, out_pose, out_l, out_f


# speedup vs baseline: 17.5519x; 1.0307x over previous
import jax
import jax.numpy as jnp
from jax.experimental import pallas as pl
from jax.experimental.pallas import tpu as pltpu

NUM_POINTS = 2_000_000
N_NEW = 1_000_000
N_FEAT = 1000
D_FEAT = 512

TILE = 128
P_LO = (N_NEW // TILE) * TILE
P_HI = P_LO + TILE

CHUNK = 8192
RING = 4


def _chunks(base, total):
    out = []
    off = base
    while off < base + total:
        sz = min(CHUNK, base + total - off)
        out.append((off, sz))
        off += sz
    return out


def _copy_body(np_ref, bnd_ref, lbnd_ref, pose_ref, nl_ref, feat_ref,
               pbuf_ref, lbuf_ref, out_p, out_pose, out_l, out_f,
               b0, b1, b2, b3, rsem, wsem, dsem):
    bufs = (b0, b1, b2, b3)

    direct = [
        pltpu.make_async_copy(nl_ref.at[pl.ds(0, P_LO)],
                              out_l.at[pl.ds(0, P_LO)], dsem),
        pltpu.make_async_copy(lbnd_ref, out_l.at[pl.ds(P_LO, TILE)], dsem),
        pltpu.make_async_copy(lbuf_ref.at[pl.ds(P_HI, NUM_POINTS - P_HI)],
                              out_l.at[pl.ds(P_HI, NUM_POINTS - P_HI)], dsem),
        pltpu.make_async_copy(bnd_ref, out_p.at[pl.ds(P_LO, TILE)], dsem),
        pltpu.make_async_copy(feat_ref, out_f, dsem),
        pltpu.make_async_copy(pose_ref, out_pose, dsem),
    ]
    for c in direct:
        c.start()

    jobs = ([(np_ref, off, sz) for off, sz in _chunks(0, P_LO)]
            + [(pbuf_ref, off, sz)
               for off, sz in _chunks(P_HI, NUM_POINTS - P_HI)])
    n = len(jobs)
    reads, writes = [], []
    for k, (src, off, sz) in enumerate(jobs):
        buf = bufs[k % RING]
        vslice = buf.at[pl.ds(0, sz)] if sz < CHUNK else buf
        reads.append(pltpu.make_async_copy(
            src.at[pl.ds(off, sz)], vslice, rsem))
        writes.append(pltpu.make_async_copy(
            vslice, out_p.at[pl.ds(off, sz)], wsem))
    for k in range(n + 1):
        if k < n:
            if k >= RING:
                writes[k - RING].wait()
            reads[k].start()
        if k >= 1:
            reads[k - 1].wait()
            writes[k - 1].start()
    for k in range(max(0, n - RING), n):
        writes[k].wait()

    for c in direct:
        c.wait()


def kernel(new_points, pose, new_point_label, new_label_feature,
           points_buf, points_label_buf, label_feature_buf, pose_buf):
    del label_feature_buf, pose_buf
    boundary = jnp.concatenate(
        [new_points[P_LO:], points_buf[N_NEW:P_HI]], axis=0)
    lboundary = jnp.concatenate(
        [new_point_label[P_LO:], points_label_buf[N_NEW:P_HI]], axis=0)
    out_p, out_pose, out_l, out_f = pl.pallas_call(
        _copy_body,
        in_specs=[pl.BlockSpec(memory_space=pl.ANY)] * 8,
        out_specs=[pl.BlockSpec(memory_space=pl.ANY)] * 4,
        out_shape=(
            jax.ShapeDtypeStruct((NUM_POINTS, 3), jnp.float32),
            jax.ShapeDtypeStruct((4, 4), jnp.float32),
            jax.ShapeDtypeStruct((NUM_POINTS,), jnp.int32),
            jax.ShapeDtypeStruct((N_FEAT, D_FEAT), jnp.float32),
        ),
        scratch_shapes=[
            pltpu.VMEM((CHUNK, 3), jnp.float32),
            pltpu.VMEM((CHUNK, 3), jnp.float32),
            pltpu.VMEM((CHUNK, 3), jnp.float32),
            pltpu.VMEM((CHUNK, 3), jnp.float32),
            pltpu.SemaphoreType.DMA,
            pltpu.SemaphoreType.DMA,
            pltpu.SemaphoreType.DMA,
        ],
    )(new_points, boundary, lboundary, pose, new_point_label,
      new_label_feature, points_buf, points_label_buf)
    return out_p, out_pose, out_l, out_f
